# Initial kernel scaffold; baseline (speedup 1.0000x reference)
import jax
import jax.numpy as jnp
from jax.experimental import pallas as pl


def kernel(max_seq_len, seq_lengths, seq_offsets, seq_embeddings, num_targets, pos_weight):
    raise NotImplementedError("write your pallas kernel here")


# speedup vs baseline: 1.3064x; 1.3064x over previous
import jax
import jax.numpy as jnp
from jax import lax
from jax.experimental import pallas as pl
from jax.experimental.pallas import tpu as pltpu
from jax.experimental.pallas import tpu_sc as plsc

_B = 8
_D = 512
_TOTAL = 15488
_NPOS = 8192
_ALPHA = float(_D) ** 0.5
_L = 16
_CHUNK = 64
_NCHUNKS = _TOTAL // _CHUNK
_NW = 32


def _body(meta_hbm, emb_hbm, pos_hbm, out_hbm,
          meta_v, idx_v, emb_v, pos_v, sem_e, sem_p):
  cid = lax.axis_index("c")
  sid = lax.axis_index("s")
  wid = sid * 2 + cid

  pltpu.sync_copy(meta_hbm, meta_v)
  off = [meta_v[b, :] for b in range(_B)]
  high = [meta_v[_B + b, :] for b in range(_B)]
  lanes = lax.iota(jnp.int32, _L)

  nloc = (_NCHUNKS - wid + _NW - 1) // _NW

  def chunk_body(i, carry):
    base = (wid + i * _NW) * _CHUNK
    cp_e = pltpu.async_copy(emb_hbm.at[pl.ds(base, _CHUNK)], emb_v, sem_e)

    for g in range(_CHUNK // _L):
      t = base + g * _L + lanes
      off_s = off[0]
      high_s = high[0]
      for b in range(1, _B):
        m = t >= off[b]
        off_s = jnp.where(m, off[b], off_s)
        high_s = jnp.where(m, high[b], high_s)
      p = jnp.minimum(t - off_s, high_s)
      p = jnp.maximum(jnp.minimum(p, _NPOS - 1), 0)
      idx_v[pl.ds(g * _L, _L)] = p

    cp_p = pltpu.async_copy(pos_hbm.at[idx_v], pos_v, sem_p)
    cp_e.wait()
    cp_p.wait()

    def row_body(r, c2):
      for k in range(_D // _L):
        sl = pl.ds(k * _L, _L)
        pos_v[r, sl] = emb_v[r, sl] * _ALPHA + pos_v[r, sl]
      return c2
    lax.fori_loop(0, _CHUNK, row_body, 0)

    pltpu.sync_copy(pos_v, out_hbm.at[pl.ds(base, _CHUNK)])
    return carry

  lax.fori_loop(0, nloc, chunk_body, 0)


def kernel(max_seq_len, seq_lengths, seq_offsets, seq_embeddings,
           num_targets, pos_weight):
  high = jnp.minimum(seq_lengths - num_targets, _NPOS - 1).astype(jnp.int32)
  meta = jnp.concatenate([
      jnp.broadcast_to(seq_offsets[:_B, None].astype(jnp.int32), (_B, _L)),
      jnp.broadcast_to(high[:, None], (_B, _L)),
  ], axis=0)

  f = pl.kernel(
      _body,
      out_type=jax.ShapeDtypeStruct((_TOTAL, _D), jnp.float32),
      mesh=plsc.VectorSubcoreMesh(core_axis_name="c", subcore_axis_name="s"),
      scratch_types=[
          pltpu.VMEM((2 * _B, _L), jnp.int32),
          pltpu.VMEM((_CHUNK,), jnp.int32),
          pltpu.VMEM((_CHUNK, _D), jnp.float32),
          pltpu.VMEM((_CHUNK, _D), jnp.float32),
          pltpu.SemaphoreType.DMA,
          pltpu.SemaphoreType.DMA,
      ],
  )
  return f(meta, seq_embeddings, pos_weight)


# speedup vs baseline: 1.7281x; 1.3228x over previous
import jax
import jax.numpy as jnp
from jax import lax
from jax.experimental import pallas as pl
from jax.experimental.pallas import tpu as pltpu
from jax.experimental.pallas import tpu_sc as plsc

_B = 8
_D = 512
_TOTAL = 15488
_NPOS = 8192
_ALPHA = float(_D) ** 0.5
_L = 16
_CHUNK = 32
_NCHUNKS = _TOTAL // _CHUNK
_NW = 32
_NMAX = -(-_NCHUNKS // _NW)


def _body(meta_hbm, emb_hbm, pos_hbm, out_hbm,
          meta_v, idx0, idx1, emb0, emb1, pos0, pos1, o0, o1,
          se0, se1, sp0, sp1, so0, so1):
  cid = lax.axis_index("c")
  sid = lax.axis_index("s")
  wid = sid * 2 + cid

  pltpu.sync_copy(meta_hbm, meta_v)
  off = [meta_v[b, :] for b in range(_B)]
  high = [meta_v[_B + b, :] for b in range(_B)]
  lanes = lax.iota(jnp.int32, _L)

  nloc = (_NCHUNKS - wid + _NW - 1) // _NW

  bufs = [(idx0, emb0, pos0, o0, se0, sp0, so0),
          (idx1, emb1, pos1, o1, se1, sp1, so1)]

  def compute_idx(base, idx_ref):
    for g in range(_CHUNK // _L):
      t = base + g * _L + lanes
      off_s = off[0]
      high_s = high[0]
      for s in range(1, _B):
        m = t >= off[s]
        off_s = jnp.where(m, off[s], off_s)
        high_s = jnp.where(m, high[s], high_s)
      p = jnp.minimum(t - off_s, high_s)
      p = jnp.maximum(jnp.minimum(p, _NPOS - 1), 0)
      idx_ref[pl.ds(g * _L, _L)] = p

  for b in range(2):
    idx_r, emb_r, pos_r, out_r, se, sp, so = bufs[b]
    base = (wid + b * _NW) * _CHUNK
    compute_idx(base, idx_r)
    pltpu.make_async_copy(emb_hbm.at[pl.ds(base, _CHUNK)], emb_r, se).start()
    pltpu.make_async_copy(pos_hbm.at[idx_r], pos_r, sp).start()

  def pair_body(i, carry):
    for b in range(2):
      slot = 2 * i + b
      idx_r, emb_r, pos_r, out_r, se, sp, so = bufs[b]

      @pl.when(slot < nloc)
      def _do(slot=slot, idx_r=idx_r, emb_r=emb_r, pos_r=pos_r, out_r=out_r,
              se=se, sp=sp, so=so):
        base = (wid + slot * _NW) * _CHUNK
        pltpu.make_async_copy(emb_hbm.at[pl.ds(base, _CHUNK)], emb_r, se).wait()
        pltpu.make_async_copy(pos_hbm.at[idx_r], pos_r, sp).wait()

        @pl.when(slot >= 2)
        def _wait_prev_store():
          prev = base - 2 * _NW * _CHUNK
          pltpu.make_async_copy(out_r, out_hbm.at[pl.ds(prev, _CHUNK)], so).wait()

        def row_body(r, c2):
          for k in range(_D // _L):
            sl = pl.ds(k * _L, _L)
            out_r[r, sl] = emb_r[r, sl] * _ALPHA + pos_r[r, sl]
          return c2
        lax.fori_loop(0, _CHUNK, row_body, 0)

        pltpu.make_async_copy(out_r, out_hbm.at[pl.ds(base, _CHUNK)], so).start()

        @pl.when(slot + 2 < nloc)
        def _prefetch():
          base2 = base + 2 * _NW * _CHUNK
          compute_idx(base2, idx_r)
          pltpu.make_async_copy(emb_hbm.at[pl.ds(base2, _CHUNK)], emb_r, se).start()
          pltpu.make_async_copy(pos_hbm.at[idx_r], pos_r, sp).start()
    return carry

  lax.fori_loop(0, (_NMAX + 1) // 2, pair_body, 0)

  for b in range(2):
    idx_r, emb_r, pos_r, out_r, se, sp, so = bufs[b]
    pltpu.make_async_copy(out_r, out_hbm.at[pl.ds(0, _CHUNK)], so).wait()


def kernel(max_seq_len, seq_lengths, seq_offsets, seq_embeddings,
           num_targets, pos_weight):
  high = jnp.minimum(seq_lengths - num_targets, _NPOS - 1).astype(jnp.int32)
  meta = jnp.concatenate([
      jnp.broadcast_to(seq_offsets[:_B, None].astype(jnp.int32), (_B, _L)),
      jnp.broadcast_to(high[:, None], (_B, _L)),
  ], axis=0)

  f = pl.kernel(
      _body,
      out_type=jax.ShapeDtypeStruct((_TOTAL, _D), jnp.float32),
      mesh=plsc.VectorSubcoreMesh(core_axis_name="c", subcore_axis_name="s"),
      scratch_types=[
          pltpu.VMEM((2 * _B, _L), jnp.int32),
          pltpu.VMEM((_CHUNK,), jnp.int32),
          pltpu.VMEM((_CHUNK,), jnp.int32),
          pltpu.VMEM((_CHUNK, _D), jnp.float32),
          pltpu.VMEM((_CHUNK, _D), jnp.float32),
          pltpu.VMEM((_CHUNK, _D), jnp.float32),
          pltpu.VMEM((_CHUNK, _D), jnp.float32),
          pltpu.VMEM((_CHUNK, _D), jnp.float32),
          pltpu.VMEM((_CHUNK, _D), jnp.float32),
          pltpu.SemaphoreType.DMA,
          pltpu.SemaphoreType.DMA,
          pltpu.SemaphoreType.DMA,
          pltpu.SemaphoreType.DMA,
          pltpu.SemaphoreType.DMA,
          pltpu.SemaphoreType.DMA,
      ],
  )
  return f(meta, seq_embeddings, pos_weight)
